# baseline (speedup 1.0000x reference)
import functools

import jax
import jax.numpy as jnp
from jax import lax
from jax.experimental import pallas as pl
from jax.experimental.pallas import tpu as pltpu
from jax.experimental.pallas import tpu_sc as plsc

NC = 1
NS = 8
NW = NC * NS
L = 16

B = 16384
BPW = B // NW
N_STEPS = 100
T_PAD = 112


def _body(t_hbm, ls_hbm, o0_hbm, o1_hbm, o2_hbm, o3_hbm,
          idx_v, ls_v, ta_v, tr_v, o0_v, o1_v, o2_v, o3_v,
          sem_idx, sem_ls, sem_out):
    wid = lax.axis_index("s") * NC + lax.axis_index("c")
    base = wid * BPW

    cp_idx = pltpu.async_copy(t_hbm.at[pl.ds(base, BPW)], idx_v, sem_idx)
    cp_ls = pltpu.async_copy(ls_hbm, ls_v.at[pl.ds(0, N_STEPS)], sem_ls)
    cp_ls.wait()

    for j in range(T_PAD // L):
        ids = lax.iota(jnp.int32, L) + (j * L)
        x = ls_v[pl.ds(j * L, L)]
        xm = plsc.load_gather(ls_v, [jnp.maximum(ids - 1, 0)])
        e = jnp.exp(x)
        em = jnp.exp(xm)
        ta_v[pl.ds(j * L, L)] = e / (1.0 + e)
        tr_v[pl.ds(j * L, L)] = jnp.exp(x - xm) * ((1.0 + em) / (1.0 + e))

    cp_idx.wait()

    n_groups = 1
    per_group = BPW // n_groups
    out_cps = []
    for g in range(n_groups):
        @plsc.parallel_loop(g * per_group, (g + 1) * per_group, L, unroll=4)
        def _chunk(off):
            sl = pl.ds(off, L)
            tv = idx_v[sl]
            a = plsc.load_gather(ta_v, [tv])
            r = plsc.load_gather(tr_v, [tv])
            o0_v[sl] = a
            o1_v[sl] = 1.0 - r
            o2_v[sl] = 1.0 - a
            o3_v[sl] = r
        gsl = pl.ds(g * per_group, per_group)
        osl = pl.ds(base + g * per_group, per_group)
        out_cps.append(pltpu.async_copy(o0_v.at[gsl], o0_hbm.at[osl], sem_out))
        out_cps.append(pltpu.async_copy(o1_v.at[gsl], o1_hbm.at[osl], sem_out))
        out_cps.append(pltpu.async_copy(o2_v.at[gsl], o2_hbm.at[osl], sem_out))
        out_cps.append(pltpu.async_copy(o3_v.at[gsl], o3_hbm.at[osl], sem_out))
    for cp in out_cps:
        cp.wait()


_sched_kernel = functools.partial(
    pl.kernel,
    out_type=tuple(jax.ShapeDtypeStruct((B,), jnp.float32) for _ in range(4)),
    mesh=plsc.VectorSubcoreMesh(
        core_axis_name="c", subcore_axis_name="s",
        num_cores=NC, num_subcores=NS),
    compiler_params=pltpu.CompilerParams(needs_layout_passes=False),
    scratch_types=[
        pltpu.VMEM((BPW,), jnp.int32),
        pltpu.VMEM((T_PAD,), jnp.float32),
        pltpu.VMEM((T_PAD,), jnp.float32),
        pltpu.VMEM((T_PAD,), jnp.float32),
        pltpu.VMEM((BPW,), jnp.float32),
        pltpu.VMEM((BPW,), jnp.float32),
        pltpu.VMEM((BPW,), jnp.float32),
        pltpu.VMEM((BPW,), jnp.float32),
        pltpu.SemaphoreType.DMA,
        pltpu.SemaphoreType.DMA,
        pltpu.SemaphoreType.DMA,
    ],
)(_body)


@jax.jit
def kernel(t, log_snr):
    return _sched_kernel(t.astype(jnp.int32), log_snr)


# speedup vs baseline: 1.0048x; 1.0048x over previous
import functools

import jax
import jax.numpy as jnp
from jax import lax
from jax.experimental import pallas as pl
from jax.experimental.pallas import tpu as pltpu
from jax.experimental.pallas import tpu_sc as plsc

NC = 1
NS = 16
NW = NC * NS
L = 16

B = 16384
BPW = B // NW
N_STEPS = 100
T_PAD = 112


def _body(t_hbm, ls_hbm, o0_hbm, o1_hbm, o2_hbm, o3_hbm,
          idx_v, ls_v, ta_v, tr_v, o0_v, o1_v, o2_v, o3_v,
          sem_idx, sem_ls, sem_out):
    wid = lax.axis_index("s") * NC + lax.axis_index("c")
    base = wid * BPW

    cp_idx = pltpu.async_copy(t_hbm.at[pl.ds(base, BPW)], idx_v, sem_idx)
    cp_ls = pltpu.async_copy(ls_hbm, ls_v.at[pl.ds(0, N_STEPS)], sem_ls)
    cp_ls.wait()

    for j in range(T_PAD // L):
        ids = lax.iota(jnp.int32, L) + (j * L)
        x = ls_v[pl.ds(j * L, L)]
        xm = plsc.load_gather(ls_v, [jnp.maximum(ids - 1, 0)])
        e = jnp.exp(x)
        em = jnp.exp(xm)
        ta_v[pl.ds(j * L, L)] = e / (1.0 + e)
        tr_v[pl.ds(j * L, L)] = jnp.exp(x - xm) * ((1.0 + em) / (1.0 + e))

    cp_idx.wait()

    n_groups = 1
    per_group = BPW // n_groups
    out_cps = []
    for g in range(n_groups):
        @plsc.parallel_loop(g * per_group, (g + 1) * per_group, L, unroll=8)
        def _chunk(off):
            sl = pl.ds(off, L)
            tv = idx_v[sl]
            a = plsc.load_gather(ta_v, [tv])
            r = plsc.load_gather(tr_v, [tv])
            o0_v[sl] = a
            o1_v[sl] = 1.0 - r
            o2_v[sl] = 1.0 - a
            o3_v[sl] = r
        gsl = pl.ds(g * per_group, per_group)
        osl = pl.ds(base + g * per_group, per_group)
        out_cps.append(pltpu.async_copy(o0_v.at[gsl], o0_hbm.at[osl], sem_out))
        out_cps.append(pltpu.async_copy(o1_v.at[gsl], o1_hbm.at[osl], sem_out))
        out_cps.append(pltpu.async_copy(o2_v.at[gsl], o2_hbm.at[osl], sem_out))
        out_cps.append(pltpu.async_copy(o3_v.at[gsl], o3_hbm.at[osl], sem_out))
    for cp in out_cps:
        cp.wait()


_sched_kernel = functools.partial(
    pl.kernel,
    out_type=tuple(jax.ShapeDtypeStruct((B,), jnp.float32) for _ in range(4)),
    mesh=plsc.VectorSubcoreMesh(
        core_axis_name="c", subcore_axis_name="s",
        num_cores=NC, num_subcores=NS),
    compiler_params=pltpu.CompilerParams(needs_layout_passes=False),
    scratch_types=[
        pltpu.VMEM((BPW,), jnp.int32),
        pltpu.VMEM((T_PAD,), jnp.float32),
        pltpu.VMEM((T_PAD,), jnp.float32),
        pltpu.VMEM((T_PAD,), jnp.float32),
        pltpu.VMEM((BPW,), jnp.float32),
        pltpu.VMEM((BPW,), jnp.float32),
        pltpu.VMEM((BPW,), jnp.float32),
        pltpu.VMEM((BPW,), jnp.float32),
        pltpu.SemaphoreType.DMA,
        pltpu.SemaphoreType.DMA,
        pltpu.SemaphoreType.DMA,
    ],
)(_body)


@jax.jit
def kernel(t, log_snr):
    return _sched_kernel(t.astype(jnp.int32), log_snr)


# speedup vs baseline: 1.0101x; 1.0053x over previous
import functools

import jax
import jax.numpy as jnp
from jax import lax
from jax.experimental import pallas as pl
from jax.experimental.pallas import tpu as pltpu
from jax.experimental.pallas import tpu_sc as plsc

NC = 1
NS = 16
NW = NC * NS
L = 16

B = 16384
BPW = B // NW
N_STEPS = 100
T_PAD = 112


def _body(t_hbm, ls_hbm, o0_hbm, o1_hbm, o2_hbm, o3_hbm,
          idx_v, ls_v, ta_v, tr_v, o0_v, o1_v, o2_v, o3_v,
          sem_idx, sem_ls, sem_out):
    wid = lax.axis_index("s") * NC + lax.axis_index("c")
    base = wid * BPW

    cp_idx = pltpu.async_copy(t_hbm.at[pl.ds(base, BPW)], idx_v, sem_idx)
    cp_ls = pltpu.async_copy(ls_hbm, ls_v.at[pl.ds(0, N_STEPS)], sem_ls)
    cp_ls.wait()

    for j in range(T_PAD // L):
        ids = lax.iota(jnp.int32, L) + (j * L)
        x = ls_v[pl.ds(j * L, L)]
        xm = plsc.load_gather(ls_v, [jnp.maximum(ids - 1, 0)])
        e = jnp.exp(x)
        em = jnp.exp(xm)
        ta_v[pl.ds(j * L, L)] = e / (1.0 + e)
        tr_v[pl.ds(j * L, L)] = jnp.exp(x - xm) * ((1.0 + em) / (1.0 + e))

    cp_idx.wait()

    n_groups = 1
    per_group = BPW // n_groups
    out_cps = []
    for g in range(n_groups):
        @plsc.parallel_loop(g * per_group, (g + 1) * per_group, L, unroll=4)
        def _chunk(off):
            sl = pl.ds(off, L)
            tv = idx_v[sl]
            a = plsc.load_gather(ta_v, [tv])
            r = plsc.load_gather(tr_v, [tv])
            o0_v[sl] = a
            o1_v[sl] = 1.0 - r
            o2_v[sl] = 1.0 - a
            o3_v[sl] = r
        gsl = pl.ds(g * per_group, per_group)
        osl = pl.ds(base + g * per_group, per_group)
        out_cps.append(pltpu.async_copy(o0_v.at[gsl], o0_hbm.at[osl], sem_out))
        out_cps.append(pltpu.async_copy(o1_v.at[gsl], o1_hbm.at[osl], sem_out))
        out_cps.append(pltpu.async_copy(o2_v.at[gsl], o2_hbm.at[osl], sem_out))
        out_cps.append(pltpu.async_copy(o3_v.at[gsl], o3_hbm.at[osl], sem_out))
    for cp in out_cps:
        cp.wait()


_sched_kernel = functools.partial(
    pl.kernel,
    out_type=tuple(jax.ShapeDtypeStruct((B,), jnp.float32) for _ in range(4)),
    mesh=plsc.VectorSubcoreMesh(
        core_axis_name="c", subcore_axis_name="s",
        num_cores=NC, num_subcores=NS),
    compiler_params=pltpu.CompilerParams(needs_layout_passes=False),
    scratch_types=[
        pltpu.VMEM((BPW,), jnp.int32),
        pltpu.VMEM((T_PAD,), jnp.float32),
        pltpu.VMEM((T_PAD,), jnp.float32),
        pltpu.VMEM((T_PAD,), jnp.float32),
        pltpu.VMEM((BPW,), jnp.float32),
        pltpu.VMEM((BPW,), jnp.float32),
        pltpu.VMEM((BPW,), jnp.float32),
        pltpu.VMEM((BPW,), jnp.float32),
        pltpu.SemaphoreType.DMA,
        pltpu.SemaphoreType.DMA,
        pltpu.SemaphoreType.DMA,
    ],
)(_body)


@jax.jit
def kernel(t, log_snr):
    return _sched_kernel(t.astype(jnp.int32), log_snr)
